# baseline (speedup 1.0000x reference)
import functools

import jax
import jax.numpy as jnp
from jax import lax
from jax.experimental import pallas as pl
from jax.experimental.pallas import tpu as pltpu
from jax.experimental.pallas import tpu_sc as plsc

SCORE_THRESHOLD = 0.05
NMS_THRESHOLD = 0.5
MAX_DETECTIONS = 100
N_BOXES = 5000
N_PAD = 5120
NT = 16
PER = N_PAD // NT
NCH = PER // 16
SENTINEL = N_PAD
G_ROWS = N_PAD + 8

NEG = float("-inf")
BIGF = 1e9


def _bmax(v, lanes):
    for s in (1, 2, 4, 8):
        p = v.at[jnp.bitwise_xor(lanes, s)].get(mode="promise_in_bounds")
        v = jnp.maximum(v, p)
    return v


def _bmin(v, lanes):
    for s in (1, 2, 4, 8):
        p = v.at[jnp.bitwise_xor(lanes, s)].get(mode="promise_in_bounds")
        v = jnp.minimum(v, p)
    return v


def _sc_body(bt_ref, cls_ref, g_ref,
             out_rows_ref, out_scores_ref, out_oris_ref, out_valid_ref,
             x1_v, y1_v, x2_v, y2_v, area_v, scores_v, ori_v, cls_v,
             pub_v, lb_v, rec_s, rec_j, rec_o,
             idx_v, rows_v, osc_v, oori_v, oval_v,
             board, sem):
    sid = lax.axis_index("s")
    base = sid * PER
    lanes = lax.iota(jnp.int32, 16)
    lanes_f = lanes.astype(jnp.float32)

    pltpu.sync_copy(bt_ref.at[pl.ds(0 * N_PAD + base, PER)], x1_v)
    pltpu.sync_copy(bt_ref.at[pl.ds(1 * N_PAD + base, PER)], y1_v)
    pltpu.sync_copy(bt_ref.at[pl.ds(2 * N_PAD + base, PER)], x2_v)
    pltpu.sync_copy(bt_ref.at[pl.ds(3 * N_PAD + base, PER)], y2_v)
    for r in range(8):
        pltpu.sync_copy(cls_ref.at[pl.ds(r * N_PAD + base, PER)],
                        cls_v.at[pl.ds(r * PER, PER)])

    for k in range(NCH):
        off = k * 16
        c0 = jnp.maximum(cls_v[pl.ds(0 * PER + off, 16)],
                         cls_v[pl.ds(4 * PER + off, 16)])
        c1 = jnp.maximum(cls_v[pl.ds(1 * PER + off, 16)],
                         cls_v[pl.ds(5 * PER + off, 16)])
        c2 = jnp.maximum(cls_v[pl.ds(2 * PER + off, 16)],
                         cls_v[pl.ds(6 * PER + off, 16)])
        c3 = jnp.maximum(cls_v[pl.ds(3 * PER + off, 16)],
                         cls_v[pl.ds(7 * PER + off, 16)])
        best = c0
        ori = jnp.zeros((16,), jnp.float32)
        for kk, c in ((1, c1), (2, c2), (3, c3)):
            upd = c > best
            ori = jnp.where(upd, float(kk), ori)
            best = jnp.maximum(best, c)
        scores_v[pl.ds(off, 16)] = jnp.where(best > SCORE_THRESHOLD, best, NEG)
        ori_v[pl.ds(off, 16)] = ori
        x1c = x1_v[pl.ds(off, 16)]
        y1c = y1_v[pl.ds(off, 16)]
        x2c = x2_v[pl.ds(off, 16)]
        y2c = y2_v[pl.ds(off, 16)]
        area_v[pl.ds(off, 16)] = (x2c - x1c) * (y2c - y1c)

    best = jnp.full((16,), NEG, jnp.float32)
    bidx = jnp.full((16,), BIGF, jnp.float32)
    for k in range(NCH):
        off = k * 16
        s = scores_v[pl.ds(off, 16)]
        gidf = (base + off).astype(jnp.float32) + lanes_f
        upd = s > best
        bidx = jnp.where(upd, gidf, bidx)
        best = jnp.maximum(best, s)

    def body(t, carry):
        count, best, bidx = carry
        m_l = _bmax(best, lanes)
        j_l = _bmin(jnp.where(best == m_l, bidx, BIGF), lanes)
        l_l = jnp.where(m_l > NEG, j_l - base.astype(jnp.float32), 0.0)
        lb = l_l.astype(jnp.int32)
        x1b = plsc.load_gather(x1_v, [lb])
        y1b = plsc.load_gather(y1_v, [lb])
        x2b = plsc.load_gather(x2_v, [lb])
        y2b = plsc.load_gather(y2_v, [lb])
        arb = plsc.load_gather(area_v, [lb])
        orb = plsc.load_gather(ori_v, [lb])
        pub = jnp.where(lanes == 0, m_l, 0.0)
        pub = jnp.where(lanes == 1, j_l, pub)
        pub = jnp.where(lanes == 2, x1b, pub)
        pub = jnp.where(lanes == 3, y1b, pub)
        pub = jnp.where(lanes == 4, x2b, pub)
        pub = jnp.where(lanes == 5, y2b, pub)
        pub = jnp.where(lanes == 6, arb, pub)
        pub = jnp.where(lanes == 7, orb, pub)
        pub_v[...] = pub
        buf = (t & 1) * 256
        pltpu.sync_copy(pub_v, board.at[pl.ds(buf + sid * 16, 16)])
        plsc.subcore_barrier()
        pltpu.sync_copy(board.at[pl.ds(buf, 256)], lb_v)

        colm = plsc.load_gather(lb_v, [lanes * 16 + 0])
        colj = plsc.load_gather(lb_v, [lanes * 16 + 1])
        m = _bmax(colm, lanes)
        jgf = _bmin(jnp.where(colm == m, colj, BIGF), lanes)
        wt = _bmin(jnp.where(colj == jgf, lanes_f, 16.0), lanes)
        wb = wt.astype(jnp.int32) * 16
        x1j = plsc.load_gather(lb_v, [wb + 2])
        y1j = plsc.load_gather(lb_v, [wb + 3])
        x2j = plsc.load_gather(lb_v, [wb + 4])
        y2j = plsc.load_gather(lb_v, [wb + 5])
        arj = plsc.load_gather(lb_v, [wb + 6])
        orj = plsc.load_gather(lb_v, [wb + 7])
        got = m > NEG

        best = jnp.full((16,), NEG, jnp.float32)
        bidx = jnp.full((16,), BIGF, jnp.float32)
        for k in range(NCH):
            off = k * 16
            x1c = x1_v[pl.ds(off, 16)]
            y1c = y1_v[pl.ds(off, 16)]
            x2c = x2_v[pl.ds(off, 16)]
            y2c = y2_v[pl.ds(off, 16)]
            arc = area_v[pl.ds(off, 16)]
            s = scores_v[pl.ds(off, 16)]
            xx1 = jnp.maximum(x1j, x1c)
            yy1 = jnp.maximum(y1j, y1c)
            xx2 = jnp.minimum(x2j, x2c)
            yy2 = jnp.minimum(y2j, y2c)
            w = jnp.maximum(0.0, xx2 - xx1)
            h = jnp.maximum(0.0, yy2 - yy1)
            inter = w * h
            iou = inter / (arj + arc - inter + 1e-9)
            gidf = (base + off).astype(jnp.float32) + lanes_f
            supp = ((iou > NMS_THRESHOLD) | (gidf == jgf)) & got
            snew = jnp.where(supp, NEG, s)
            scores_v[pl.ds(off, 16)] = snew
            upd = snew > best
            bidx = jnp.where(upd, gidf, bidx)
            best = jnp.maximum(best, snew)

        @pl.when(sid == 0)
        def _():
            tv = jnp.full((16,), t, jnp.int32)
            recm = (lanes == 0) & got
            plsc.store_scatter(rec_s, [tv], m, mask=recm)
            plsc.store_scatter(rec_j, [tv], jgf, mask=recm)
            plsc.store_scatter(rec_o, [tv], orj, mask=recm)

        return count + got.astype(jnp.int32), best, bidx

    count, _, _ = lax.fori_loop(0, MAX_DETECTIONS, body,
                                (jnp.zeros((16,), jnp.int32), best, bidx))

    @pl.when(sid == 0)
    def _():
        for k in range(8):
            off = k * 16
            slot = off + lanes
            validm = slot < count
            sc = rec_s[pl.ds(off, 16)]
            jf = rec_j[pl.ds(off, 16)]
            orf = rec_o[pl.ds(off, 16)]
            osc_v[pl.ds(off, 16)] = jnp.where(validm, sc, -1.0)
            oori_v[pl.ds(off, 16)] = jnp.where(
                validm, orf, -1.0).astype(jnp.int32)
            oval_v[pl.ds(off, 16)] = validm.astype(jnp.int32)
            idx_v[pl.ds(off, 16)] = jnp.where(
                validm, jf.astype(jnp.int32), SENTINEL)
        pltpu.async_copy(g_ref.at[idx_v], rows_v, sem).wait()
        pltpu.sync_copy(rows_v, out_rows_ref)
        pltpu.sync_copy(osc_v, out_scores_ref)
        pltpu.sync_copy(oori_v, out_oris_ref)
        pltpu.sync_copy(oval_v, out_valid_ref)


@jax.jit
def _run(bt, clsT, g):
    f32 = jnp.float32
    i32 = jnp.int32
    mesh = plsc.VectorSubcoreMesh(core_axis_name="c", subcore_axis_name="s")
    kfn = functools.partial(
        pl.kernel,
        out_type=(
            jax.ShapeDtypeStruct((128, 128), f32),
            jax.ShapeDtypeStruct((128,), f32),
            jax.ShapeDtypeStruct((128,), i32),
            jax.ShapeDtypeStruct((128,), i32),
        ),
        mesh=mesh,
        compiler_params=pltpu.CompilerParams(needs_layout_passes=False),
        scratch_types=[
            pltpu.VMEM((PER,), f32),
            pltpu.VMEM((PER,), f32),
            pltpu.VMEM((PER,), f32),
            pltpu.VMEM((PER,), f32),
            pltpu.VMEM((PER,), f32),
            pltpu.VMEM((PER,), f32),
            pltpu.VMEM((PER,), f32),
            pltpu.VMEM((8 * PER,), f32),
            pltpu.VMEM((16,), f32),
            pltpu.VMEM((16 * 16,), f32),
            pltpu.VMEM((128,), f32),
            pltpu.VMEM((128,), f32),
            pltpu.VMEM((128,), f32),
            pltpu.VMEM((128,), i32),
            pltpu.VMEM((128, 128), f32),
            pltpu.VMEM((128,), f32),
            pltpu.VMEM((128,), i32),
            pltpu.VMEM((128,), i32),
            pltpu.VMEM_SHARED((2 * 16 * 16,), f32),
            pltpu.SemaphoreType.DMA,
        ],
    )
    return kfn(_sc_body)(bt, clsT, g)


def kernel(boxes, dimensions, classification):
    f32 = jnp.float32
    b4 = jnp.pad(boxes[:, :4], ((0, N_PAD - N_BOXES), (0, 0)))
    bt = b4.T.reshape(-1)
    clsT = jnp.pad(classification,
                   ((0, N_PAD - N_BOXES), (0, 0))).T.reshape(-1)
    g = jnp.concatenate(
        [boxes, dimensions, jnp.zeros((N_BOXES, 1), f32)], axis=1)
    g = jnp.pad(g, ((0, G_ROWS - N_BOXES), (0, 0)), constant_values=-1.0)
    g = g.at[N_BOXES:N_PAD].set(0.0)
    g = jnp.pad(g, ((0, 0), (0, 112)))

    out_rows, out_scores, out_oris, out_valid = _run(bt, clsT, g)

    valid = out_valid[:MAX_DETECTIONS] > 0
    out_boxes = out_rows[:MAX_DETECTIONS, :12]
    out_dims = out_rows[:MAX_DETECTIONS, 12:15]
    out_s = out_scores[:MAX_DETECTIONS]
    out_labels = jnp.where(valid, 0, -1)
    out_o = out_oris[:MAX_DETECTIONS]
    return (jnp.asarray(out_boxes, dtype=jnp.float32),
            jnp.asarray(out_dims, dtype=jnp.float32),
            jnp.asarray(out_s, dtype=jnp.float32),
            jnp.asarray(out_labels, dtype=jnp.int64),
            jnp.asarray(out_o, dtype=jnp.int64))


# speedup vs baseline: 1.0447x; 1.0447x over previous
import functools

import jax
import jax.numpy as jnp
from jax import lax
from jax.experimental import pallas as pl
from jax.experimental.pallas import tpu as pltpu
from jax.experimental.pallas import tpu_sc as plsc

SCORE_THRESHOLD = 0.05
NMS_THRESHOLD = 0.5
MAX_DETECTIONS = 100
N_BOXES = 5000
N_PAD = 5120
NT = 16
PER = N_PAD // NT
NCH = PER // 16

NEG = float("-inf")
BIGF = 1e9


def _bmax(v, lanes):
    for s in (1, 2, 4, 8):
        p = v.at[jnp.bitwise_xor(lanes, s)].get(mode="promise_in_bounds")
        v = jnp.maximum(v, p)
    return v


def _bmin(v, lanes):
    for s in (1, 2, 4, 8):
        p = v.at[jnp.bitwise_xor(lanes, s)].get(mode="promise_in_bounds")
        v = jnp.minimum(v, p)
    return v


def _sc_body(bt_ref, cls_ref, g_ref,
             out_rows_ref, out_scores_ref, out_oris_ref, out_valid_ref,
             bxy_v, g_v, cls_v, area_v, scores_v, ori_v,
             pub_v, lb_v, red_v, grow_v, rows_v, rec_s, rec_o,
             osc_v, oori_v, oval_v,
             board, pick_board):
    cid = lax.axis_index("c")

    @pl.when(cid == 0)
    def _():
        sid = lax.axis_index("s")
        base = sid * PER
        basef = base.astype(jnp.float32)
        lanes = lax.iota(jnp.int32, 16)
        lanes_f = lanes.astype(jnp.float32)

        pltpu.sync_copy(bt_ref.at[pl.ds(sid * (4 * PER), 4 * PER)], bxy_v)
        pltpu.sync_copy(g_ref.at[pl.ds(sid * (16 * PER), 16 * PER)], g_v)
        pltpu.sync_copy(cls_ref.at[pl.ds(sid * (8 * PER), 8 * PER)], cls_v)

        for k in range(NCH):
            off = k * 16
            c0 = jnp.maximum(cls_v[pl.ds(0 * PER + off, 16)],
                             cls_v[pl.ds(4 * PER + off, 16)])
            c1 = jnp.maximum(cls_v[pl.ds(1 * PER + off, 16)],
                             cls_v[pl.ds(5 * PER + off, 16)])
            c2 = jnp.maximum(cls_v[pl.ds(2 * PER + off, 16)],
                             cls_v[pl.ds(6 * PER + off, 16)])
            c3 = jnp.maximum(cls_v[pl.ds(3 * PER + off, 16)],
                             cls_v[pl.ds(7 * PER + off, 16)])
            best = c0
            ori = jnp.zeros((16,), jnp.float32)
            for kk, c in ((1, c1), (2, c2), (3, c3)):
                upd = c > best
                ori = jnp.where(upd, float(kk), ori)
                best = jnp.maximum(best, c)
            scores_v[pl.ds(off, 16)] = jnp.where(
                best > SCORE_THRESHOLD, best, NEG)
            ori_v[pl.ds(off, 16)] = ori
            x1c = bxy_v[pl.ds(0 * PER + off, 16)]
            y1c = bxy_v[pl.ds(1 * PER + off, 16)]
            x2c = bxy_v[pl.ds(2 * PER + off, 16)]
            y2c = bxy_v[pl.ds(3 * PER + off, 16)]
            area_v[pl.ds(off, 16)] = (x2c - x1c) * (y2c - y1c)

        @pl.when(sid == 0)
        def _():
            neg1 = jnp.full((16,), -1.0, jnp.float32)
            for k in range(128):
                rows_v[pl.ds(k * 16, 16)] = neg1
            pltpu.sync_copy(rows_v, pick_board)

        best = jnp.full((16,), NEG, jnp.float32)
        bidx = jnp.full((16,), BIGF, jnp.float32)
        for k in range(NCH):
            off = k * 16
            s = scores_v[pl.ds(off, 16)]
            gidf = (base + off).astype(jnp.float32) + lanes_f
            upd = s > best
            bidx = jnp.where(upd, gidf, bidx)
            best = jnp.maximum(best, s)

        plsc.subcore_barrier()

        def body(t, carry):
            count, best, bidx = carry
            m_l = _bmax(best, lanes)
            j_l = _bmin(jnp.where(best == m_l, bidx, BIGF), lanes)
            l_l = jnp.where(m_l > NEG, j_l - basef, 0.0)
            lb = l_l.astype(jnp.int32)
            x1b = plsc.load_gather(bxy_v, [lb])
            y1b = plsc.load_gather(bxy_v, [lb + PER])
            x2b = plsc.load_gather(bxy_v, [lb + 2 * PER])
            y2b = plsc.load_gather(bxy_v, [lb + 3 * PER])
            arb = plsc.load_gather(area_v, [lb])
            orb = plsc.load_gather(ori_v, [lb])
            pub = jnp.where(lanes == 0, m_l, 0.0)
            pub = jnp.where(lanes == 1, j_l, pub)
            pub = jnp.where(lanes == 2, x1b, pub)
            pub = jnp.where(lanes == 3, y1b, pub)
            pub = jnp.where(lanes == 4, x2b, pub)
            pub = jnp.where(lanes == 5, y2b, pub)
            pub = jnp.where(lanes == 6, arb, pub)
            pub = jnp.where(lanes == 7, orb, pub)
            pub_v[...] = pub
            buf = (t & 1) * 256
            pltpu.sync_copy(pub_v, board.at[pl.ds(buf + sid * 16, 16)])
            plsc.subcore_barrier()
            pltpu.sync_copy(board.at[pl.ds(buf, 256)], lb_v)

            colm = plsc.load_gather(lb_v, [lanes * 16 + 0])
            colj = plsc.load_gather(lb_v, [lanes * 16 + 1])
            m = _bmax(colm, lanes)
            jgf = _bmin(jnp.where(colm == m, colj, BIGF), lanes)
            wt = _bmin(jnp.where(colj == jgf, lanes_f, 16.0), lanes)
            wb = wt.astype(jnp.int32) * 16
            x1j = plsc.load_gather(lb_v, [wb + 2])
            y1j = plsc.load_gather(lb_v, [wb + 3])
            x2j = plsc.load_gather(lb_v, [wb + 4])
            y2j = plsc.load_gather(lb_v, [wb + 5])
            arj = plsc.load_gather(lb_v, [wb + 6])
            orj = plsc.load_gather(lb_v, [wb + 7])
            got = m > NEG

            mine = got & (jgf >= basef) & (jgf < basef + float(PER))
            l_w = jnp.clip((jgf - basef), 0.0, float(PER - 1)).astype(
                jnp.int32)
            grow_v[...] = plsc.load_gather(g_v, [l_w * 16 + lanes])
            mine_s = jnp.where(mine, 1.0, 0.0)[0] > 0.5

            @pl.when(mine_s)
            def _():
                pltpu.sync_copy(grow_v, pick_board.at[pl.ds(t * 16, 16)])

            best = jnp.full((16,), NEG, jnp.float32)
            bidx = jnp.full((16,), BIGF, jnp.float32)
            for k in range(NCH):
                off = k * 16
                x1c = bxy_v[pl.ds(0 * PER + off, 16)]
                y1c = bxy_v[pl.ds(1 * PER + off, 16)]
                x2c = bxy_v[pl.ds(2 * PER + off, 16)]
                y2c = bxy_v[pl.ds(3 * PER + off, 16)]
                arc = area_v[pl.ds(off, 16)]
                s = scores_v[pl.ds(off, 16)]
                xx1 = jnp.maximum(x1j, x1c)
                yy1 = jnp.maximum(y1j, y1c)
                xx2 = jnp.minimum(x2j, x2c)
                yy2 = jnp.minimum(y2j, y2c)
                w = jnp.maximum(0.0, xx2 - xx1)
                h = jnp.maximum(0.0, yy2 - yy1)
                inter = w * h
                iou = inter / (arj + arc - inter + 1e-9)
                gidf = (base + off).astype(jnp.float32) + lanes_f
                supp = ((iou > NMS_THRESHOLD) | (gidf == jgf)) & got
                snew = jnp.where(supp, NEG, s)
                scores_v[pl.ds(off, 16)] = snew
                upd = snew > best
                bidx = jnp.where(upd, gidf, bidx)
                best = jnp.maximum(best, snew)

            @pl.when(sid == 0)
            def _():
                tv = jnp.full((16,), t, jnp.int32)
                recm = (lanes == 0) & got
                plsc.store_scatter(rec_s, [tv], m, mask=recm)
                plsc.store_scatter(rec_o, [tv], orj, mask=recm)

            return count + got.astype(jnp.int32), best, bidx

        count, _, _ = lax.fori_loop(
            0, MAX_DETECTIONS, body,
            (jnp.zeros((16,), jnp.int32), best, bidx))

        plsc.subcore_barrier()

        @pl.when(sid == 0)
        def _():
            for k in range(8):
                off = k * 16
                slot = off + lanes
                validm = slot < count
                sc = rec_s[pl.ds(off, 16)]
                orf = rec_o[pl.ds(off, 16)]
                osc_v[pl.ds(off, 16)] = jnp.where(validm, sc, -1.0)
                oori_v[pl.ds(off, 16)] = jnp.where(
                    validm, orf, -1.0).astype(jnp.int32)
                oval_v[pl.ds(off, 16)] = validm.astype(jnp.int32)
            pltpu.sync_copy(pick_board, rows_v)
            pltpu.sync_copy(rows_v, out_rows_ref)
            pltpu.sync_copy(osc_v, out_scores_ref)
            pltpu.sync_copy(oori_v, out_oris_ref)
            pltpu.sync_copy(oval_v, out_valid_ref)


@jax.jit
def _run(bt, clsT, g):
    f32 = jnp.float32
    i32 = jnp.int32
    mesh = plsc.VectorSubcoreMesh(core_axis_name="c", subcore_axis_name="s")
    kfn = functools.partial(
        pl.kernel,
        out_type=(
            jax.ShapeDtypeStruct((2048,), f32),
            jax.ShapeDtypeStruct((128,), f32),
            jax.ShapeDtypeStruct((128,), i32),
            jax.ShapeDtypeStruct((128,), i32),
        ),
        mesh=mesh,
        compiler_params=pltpu.CompilerParams(needs_layout_passes=False),
        scratch_types=[
            pltpu.VMEM((4 * PER,), f32),
            pltpu.VMEM((16 * PER,), f32),
            pltpu.VMEM((8 * PER,), f32),
            pltpu.VMEM((PER,), f32),
            pltpu.VMEM((PER,), f32),
            pltpu.VMEM((PER,), f32),
            pltpu.VMEM((16,), f32),
            pltpu.VMEM((16 * 16,), f32),
            pltpu.VMEM((16,), f32),
            pltpu.VMEM((16,), f32),
            pltpu.VMEM((2048,), f32),
            pltpu.VMEM((128,), f32),
            pltpu.VMEM((128,), f32),
            pltpu.VMEM((128,), f32),
            pltpu.VMEM((128,), i32),
            pltpu.VMEM((128,), i32),
            pltpu.VMEM_SHARED((2 * 16 * 16,), f32),
            pltpu.VMEM_SHARED((2048,), f32),
        ],
    )
    return kfn(_sc_body)(bt, clsT, g)


def kernel(boxes, dimensions, classification):
    f32 = jnp.float32
    b4 = jnp.pad(boxes[:, :4], ((0, N_PAD - N_BOXES), (0, 0)))
    bt = b4.reshape(NT, PER, 4).transpose(0, 2, 1).reshape(-1)
    clsT = jnp.pad(classification, ((0, N_PAD - N_BOXES), (0, 0)))
    clsT = clsT.reshape(NT, PER, 8).transpose(0, 2, 1).reshape(-1)
    g = jnp.concatenate(
        [boxes, dimensions, jnp.zeros((N_BOXES, 1), f32)], axis=1)
    g = jnp.pad(g, ((0, N_PAD - N_BOXES), (0, 0))).reshape(-1)

    out_rows, out_scores, out_oris, out_valid = _run(bt, clsT, g)
    out_rows = out_rows.reshape(128, 16)

    valid = out_valid[:MAX_DETECTIONS] > 0
    out_boxes = out_rows[:MAX_DETECTIONS, :12]
    out_dims = out_rows[:MAX_DETECTIONS, 12:15]
    out_s = out_scores[:MAX_DETECTIONS]
    out_labels = jnp.where(valid, 0, -1)
    out_o = out_oris[:MAX_DETECTIONS]
    return (jnp.asarray(out_boxes, dtype=jnp.float32),
            jnp.asarray(out_dims, dtype=jnp.float32),
            jnp.asarray(out_s, dtype=jnp.float32),
            jnp.asarray(out_labels, dtype=jnp.int64),
            jnp.asarray(out_o, dtype=jnp.int64))
